# Initial kernel scaffold; baseline (speedup 1.0000x reference)
import jax
import jax.numpy as jnp
from jax.experimental import pallas as pl


def kernel(x, position_weight, value_weight):
    raise NotImplementedError("write your pallas kernel here")


# speedup vs baseline: 2.0261x; 2.0261x over previous
import functools

import jax
import jax.numpy as jnp
from jax import lax
from jax.experimental import pallas as pl
from jax.experimental.pallas import tpu as pltpu
from jax.experimental.pallas import tpu_sc as plsc

_D = 4096
_P = 784
_L = 256
_B = 32
_W = _D // 32
_WH = _W // 2
_G = _P // 16
_HALF_P = _P // 2


def _pack_bits(w):
    bits = (w < 0).astype(jnp.uint32).reshape(w.shape[0], _W, 32)
    shifts = jnp.arange(32, dtype=jnp.uint32)
    words = jnp.sum(bits << shifts, axis=-1).astype(jnp.uint32)
    return lax.bitcast_convert_type(words, jnp.int32)


def _sc_body(x_hbm, posbt_hbm, valb_hbm, out_hbm,
             xv, idxv, valb_v, post_v, part_v, out_v):
    b = lax.axis_index("s") * 2 + lax.axis_index("c")

    pltpu.sync_copy(x_hbm.at[b], xv)
    pltpu.sync_copy(valb_hbm, valb_v)

    for g in range(_G):
        v = xv[pl.ds(g * 16, 16)] * jnp.float32(_L - 1)
        t = v.astype(jnp.int32)
        frac = v - t.astype(jnp.float32)
        half = jnp.float32(0.5)
        inc = jnp.where(
            (frac > half) | ((frac == half) & ((t & 1) == 1)), 1, 0)
        idx = jnp.minimum(jnp.maximum(t + inc, 0), _L - 1)
        idxv[pl.ds(g * 16, 16)] = idx

    lanes = lax.iota(jnp.int32, 16)

    for half in range(2):
        pltpu.sync_copy(posbt_hbm.at[pl.ds(half * _WH, _WH)], post_v)

        def w_body(wl, _, half=half):
            w_glob = half * _WH + wl
            planes = [jnp.zeros((16,), jnp.uint32) for _ in range(6)]
            for g in range(_G):
                ig = idxv[pl.ds(g * 16, 16)]
                vw = plsc.load_gather(valb_v, [ig * _W + w_glob])
                pw = post_v[wl, pl.ds(g * 16, 16)]
                carry = lax.bitcast_convert_type(vw ^ pw, jnp.uint32)
                for k in range(6):
                    t = planes[k] ^ carry
                    carry = planes[k] & carry
                    planes[k] = t
            for j in range(32):
                cnt = jnp.zeros((16,), jnp.uint32)
                for k in range(6):
                    cnt = cnt + (((planes[k] >> j) & jnp.uint32(1)) << k)
                part_v[pl.ds(j * 16, 16)] = cnt.astype(jnp.int32)
            for s in range(2):
                rows = (lanes + (s * 16)) * 16
                n = jnp.zeros((16,), jnp.int32)
                for ln in range(16):
                    n = n + plsc.load_gather(part_v, [rows + ln])
                out = jnp.where(n < _HALF_P, jnp.float32(1.0),
                                jnp.float32(-1.0))
                out_v[pl.ds(w_glob * 32 + s * 16, 16)] = out
            return 0

        lax.fori_loop(0, _WH, w_body, 0)

    pltpu.sync_copy(out_v, out_hbm.at[b])


def kernel(x, position_weight, value_weight):
    xf = x.reshape(_B, _P)
    posbits_t = _pack_bits(position_weight).T.reshape(_W, _P)
    valbits = _pack_bits(value_weight).reshape(_L * _W)

    mesh = plsc.VectorSubcoreMesh(core_axis_name="c", subcore_axis_name="s")
    f = functools.partial(
        pl.kernel,
        out_type=jax.ShapeDtypeStruct((_B, _D), jnp.float32),
        mesh=mesh,
        compiler_params=pltpu.CompilerParams(needs_layout_passes=False),
        scratch_types=[
            pltpu.VMEM((_P,), jnp.float32),
            pltpu.VMEM((_P,), jnp.int32),
            pltpu.VMEM((_L * _W,), jnp.int32),
            pltpu.VMEM((_WH, _P), jnp.int32),
            pltpu.VMEM((32 * 16,), jnp.int32),
            pltpu.VMEM((_D,), jnp.float32),
        ],
    )(_sc_body)
    return f(xf, posbits_t, valbits)


# speedup vs baseline: 3.3681x; 1.6624x over previous
import functools

import jax
import jax.numpy as jnp
from jax import lax
from jax.experimental import pallas as pl
from jax.experimental.pallas import tpu as pltpu
from jax.experimental.pallas import tpu_sc as plsc

_D = 4096
_P = 784
_L = 256
_B = 32
_W = _D // 32
_PH = _P // 2
_NWG = _W // 16
_CH = 49
_NCH = _PH // _CH
_HALF_P = _P // 2
_PLANES = 10


def _pack_bits(w):
    bits = (w < 0).astype(jnp.uint32).reshape(w.shape[0], _W, 32)
    shifts = jnp.arange(32, dtype=jnp.uint32)
    words = jnp.sum(bits << shifts, axis=-1).astype(jnp.uint32)
    return lax.bitcast_convert_type(words, jnp.int32)


def _csa(a, b, c):
    t = a ^ b
    return t ^ c, (a & b) | (c & t)


def _reduce_chunk(words):
    pools = {}

    def push(k, x):
        pool = pools.setdefault(k, [])
        pool.append(x)
        if len(pool) >= 3:
            s, cy = _csa(pool.pop(), pool.pop(), pool.pop())
            pool.append(s)
            push(k + 1, cy)

    for x in words:
        push(0, x)
    for k in sorted(pools):
        pool = pools[k]
        while len(pool) > 1:
            a, b = pool.pop(), pool.pop()
            pool.append(a ^ b)
            push(k + 1, a & b)
    return {k: p[0] for k, p in pools.items() if p}


def _merge_planes(planes, addend):
    carry = None
    out = list(planes)
    for k in range(_PLANES):
        q = addend.get(k)
        if carry is None and q is None:
            continue
        if carry is None:
            out[k], carry = out[k] ^ q, out[k] & q
        elif q is None:
            out[k], carry = out[k] ^ carry, out[k] & carry
        else:
            out[k], carry = _csa(out[k], q, carry)
    return out


def _sc_body(x_hbm, posb_hbm, valb_hbm, out_hbm, xv, idxv, valb_v, posb_v,
             planes_v, out_v):
    b = lax.axis_index("s") * 2 + lax.axis_index("c")

    pltpu.sync_copy(x_hbm.at[b], xv)
    pltpu.sync_copy(valb_hbm, valb_v)

    for g in range(_P // 16):
        v = xv[pl.ds(g * 16, 16)] * jnp.float32(_L - 1)
        t = v.astype(jnp.int32)
        frac = v - t.astype(jnp.float32)
        half = jnp.float32(0.5)
        inc = jnp.where(
            (frac > half) | ((frac == half) & ((t & 1) == 1)), 1, 0)
        idx = jnp.minimum(jnp.maximum(t + inc, 0), _L - 1)
        idxv[pl.ds(g * 16, 16)] = idx

    lane32 = lax.iota(jnp.int32, 16) * 32
    zero = jnp.zeros((16,), jnp.uint32)

    for ph in range(2):
        pltpu.sync_copy(posb_hbm.at[pl.ds(ph * _PH, _PH)], posb_v)

        def wg_body(wg, _, ph=ph):
            woff = wg * 16

            if ph == 0:
                planes0 = tuple(zero for _ in range(_PLANES))
            else:
                planes0 = tuple(
                    lax.bitcast_convert_type(
                        planes_v[pl.ds((wg * _PLANES + k) * 16, 16)],
                        jnp.uint32)
                    for k in range(_PLANES))

            def chunk_body(ci, planes):
                planes = list(planes)
                base = ci * _CH
                iv = [idxv[pl.ds(ph * _PH + base + o, 16)]
                      for o in (0, 16, 32, 33)]
                words = []
                for i in range(_CH):
                    l = iv[i // 16][i % 16] if i < 48 else iv[3][15]
                    vw = valb_v[l, pl.ds(woff, 16)]
                    pw = posb_v[base + i, pl.ds(woff, 16)]
                    words.append(
                        lax.bitcast_convert_type(vw ^ pw, jnp.uint32))
                return tuple(_merge_planes(planes, _reduce_chunk(words)))

            planes = lax.fori_loop(0, _NCH, chunk_body, planes0)

            if ph == 0:
                for k in range(_PLANES):
                    planes_v[pl.ds((wg * _PLANES + k) * 16, 16)] = (
                        lax.bitcast_convert_type(planes[k], jnp.int32))
            else:
                dbase = lane32 + (wg * 512)
                for j in range(32):
                    cnt = jnp.zeros((16,), jnp.uint32)
                    for k in range(_PLANES):
                        cnt = cnt + (((planes[k] >> j) & jnp.uint32(1)) << k)
                    out = jnp.where(cnt < jnp.uint32(_HALF_P),
                                    jnp.float32(1.0), jnp.float32(-1.0))
                    plsc.store_scatter(out_v, [dbase + j], out)
            return 0

        lax.fori_loop(0, _NWG, wg_body, 0)

    pltpu.sync_copy(out_v, out_hbm.at[b])


def kernel(x, position_weight, value_weight):
    xf = x.reshape(_B, _P)
    posbits = _pack_bits(position_weight)
    valbits = _pack_bits(value_weight)

    mesh = plsc.VectorSubcoreMesh(core_axis_name="c", subcore_axis_name="s")
    f = functools.partial(
        pl.kernel,
        out_type=jax.ShapeDtypeStruct((_B, _D), jnp.float32),
        mesh=mesh,
        compiler_params=pltpu.CompilerParams(needs_layout_passes=False),
        scratch_types=[
            pltpu.VMEM((_P,), jnp.float32),
            pltpu.VMEM((_P,), jnp.int32),
            pltpu.VMEM((_L, _W), jnp.int32),
            pltpu.VMEM((_PH, _W), jnp.int32),
            pltpu.VMEM((_NWG * _PLANES * 16,), jnp.int32),
            pltpu.VMEM((_D,), jnp.float32),
        ],
    )(_sc_body)
    return f(xf, posbits, valbits)


# speedup vs baseline: 3.7888x; 1.1249x over previous
import functools

import jax
import jax.numpy as jnp
from jax import lax
from jax.experimental import pallas as pl
from jax.experimental.pallas import tpu as pltpu
from jax.experimental.pallas import tpu_sc as plsc

_D = 4096
_P = 784
_L = 256
_B = 32
_W = _D // 32
_PH = _P // 2
_NWG = _W // 16
_CH = 49
_NCH = _PH // _CH
_HALF_P = _P // 2
_PLANES = 10


def _pack_bits(w):
    u = lax.bitcast_convert_type(w, jnp.uint32) >> 31
    acc = u[:, : _W]
    for j in range(1, 32):
        acc = acc | (u[:, j * _W:(j + 1) * _W] << j)
    return lax.bitcast_convert_type(acc, jnp.int32)


def _csa(a, b, c):
    t = a ^ b
    return t ^ c, (a & b) | (c & t)


def _reduce_chunk(words):
    pools = {}

    def push(k, x):
        pool = pools.setdefault(k, [])
        pool.append(x)
        if len(pool) >= 3:
            s, cy = _csa(pool.pop(), pool.pop(), pool.pop())
            pool.append(s)
            push(k + 1, cy)

    for x in words:
        push(0, x)
    for k in sorted(pools):
        pool = pools[k]
        while len(pool) > 1:
            a, b = pool.pop(), pool.pop()
            pool.append(a ^ b)
            push(k + 1, a & b)
    return {k: p[0] for k, p in pools.items() if p}


def _merge_planes(planes, addend):
    carry = None
    out = list(planes)
    for k in range(_PLANES):
        q = addend.get(k)
        if carry is None and q is None:
            continue
        if carry is None:
            out[k], carry = out[k] ^ q, out[k] & q
        elif q is None:
            out[k], carry = out[k] ^ carry, out[k] & carry
        else:
            out[k], carry = _csa(out[k], q, carry)
    return out


def _sc_body(x_hbm, posb_hbm, valb_hbm, out_hbm, xv, idxv, valb_v, posb_v,
             planes_v, out_v):
    b = lax.axis_index("s") * 2 + lax.axis_index("c")

    pltpu.sync_copy(x_hbm.at[b], xv)
    pltpu.sync_copy(valb_hbm, valb_v)

    for g in range(_P // 16):
        v = xv[pl.ds(g * 16, 16)] * jnp.float32(_L - 1)
        t = v.astype(jnp.int32)
        frac = v - t.astype(jnp.float32)
        half = jnp.float32(0.5)
        inc = jnp.where(
            (frac > half) | ((frac == half) & ((t & 1) == 1)), 1, 0)
        idx = jnp.minimum(jnp.maximum(t + inc, 0), _L - 1)
        idxv[pl.ds(g * 16, 16)] = idx

    zero = jnp.zeros((16,), jnp.uint32)

    for ph in range(2):
        pltpu.sync_copy(posb_hbm.at[pl.ds(ph * _PH, _PH)], posb_v)

        def wg_body(wg, _, ph=ph):
            woff = wg * 16

            if ph == 0:
                planes0 = tuple(zero for _ in range(_PLANES))
            else:
                planes0 = tuple(
                    lax.bitcast_convert_type(
                        planes_v[pl.ds((wg * _PLANES + k) * 16, 16)],
                        jnp.uint32)
                    for k in range(_PLANES))

            def chunk_body(ci, planes):
                planes = list(planes)
                base = ci * _CH
                iv = [idxv[pl.ds(ph * _PH + base + o, 16)]
                      for o in (0, 16, 32, 33)]
                words = []
                for i in range(_CH):
                    l = iv[i // 16][i % 16] if i < 48 else iv[3][15]
                    vw = valb_v[l, pl.ds(woff, 16)]
                    pw = posb_v[base + i, pl.ds(woff, 16)]
                    words.append(
                        lax.bitcast_convert_type(vw ^ pw, jnp.uint32))
                return tuple(_merge_planes(planes, _reduce_chunk(words)))

            planes = lax.fori_loop(0, _NCH, chunk_body, planes0)

            if ph == 0:
                for k in range(_PLANES):
                    planes_v[pl.ds((wg * _PLANES + k) * 16, 16)] = (
                        lax.bitcast_convert_type(planes[k], jnp.int32))
            else:
                for j in range(32):
                    cnt = jnp.zeros((16,), jnp.uint32)
                    for k in range(_PLANES):
                        cnt = cnt + (((planes[k] >> j) & jnp.uint32(1)) << k)
                    out = jnp.where(cnt < jnp.uint32(_HALF_P),
                                    jnp.float32(1.0), jnp.float32(-1.0))
                    out_v[pl.ds(j * _W + wg * 16, 16)] = out
            return 0

        lax.fori_loop(0, _NWG, wg_body, 0)

    pltpu.sync_copy(out_v, out_hbm.at[b])


def kernel(x, position_weight, value_weight):
    xf = x.reshape(_B, _P)
    posbits = _pack_bits(position_weight)
    valbits = _pack_bits(value_weight)

    mesh = plsc.VectorSubcoreMesh(core_axis_name="c", subcore_axis_name="s")
    f = functools.partial(
        pl.kernel,
        out_type=jax.ShapeDtypeStruct((_B, _D), jnp.float32),
        mesh=mesh,
        compiler_params=pltpu.CompilerParams(needs_layout_passes=False),
        scratch_types=[
            pltpu.VMEM((_P,), jnp.float32),
            pltpu.VMEM((_P,), jnp.int32),
            pltpu.VMEM((_L, _W), jnp.int32),
            pltpu.VMEM((_PH, _W), jnp.int32),
            pltpu.VMEM((_NWG * _PLANES * 16,), jnp.int32),
            pltpu.VMEM((_D,), jnp.float32),
        ],
    )(_sc_body)
    return f(xf, posbits, valbits)


# speedup vs baseline: 3.8049x; 1.0043x over previous
import functools

import jax
import jax.numpy as jnp
from jax import lax
from jax.experimental import pallas as pl
from jax.experimental.pallas import tpu as pltpu
from jax.experimental.pallas import tpu_sc as plsc

_D = 4096
_P = 784
_L = 256
_B = 32
_W = _D // 32
_PH0, _PH1 = 400, 384
_NWG = _W // 16
_NCH = 8
_HALF_P = _P // 2
_PLANES = 10


def _pack_bits(w):
    u = lax.bitcast_convert_type(w, jnp.uint32) >> 31
    acc = u[:, : _W]
    for j in range(1, 32):
        acc = acc | (u[:, j * _W:(j + 1) * _W] << j)
    return lax.bitcast_convert_type(acc, jnp.int32)


def _csa(a, b, c):
    t = a ^ b
    return t ^ c, (a & b) | (c & t)


def _reduce_chunk(words):
    pools = {}

    def push(k, x):
        pool = pools.setdefault(k, [])
        pool.append(x)
        if len(pool) >= 3:
            s, cy = _csa(pool.pop(), pool.pop(), pool.pop())
            pool.append(s)
            push(k + 1, cy)

    for x in words:
        push(0, x)
    for k in sorted(pools):
        pool = pools[k]
        while len(pool) > 1:
            a, b = pool.pop(), pool.pop()
            pool.append(a ^ b)
            push(k + 1, a & b)
    return {k: p[0] for k, p in pools.items() if p}


def _merge_planes(planes, addend):
    carry = None
    out = list(planes)
    for k in range(_PLANES):
        q = addend.get(k)
        if carry is None and q is None:
            continue
        if carry is None:
            out[k], carry = out[k] ^ q, out[k] & q
        elif q is None:
            out[k], carry = out[k] ^ carry, out[k] & carry
        else:
            out[k], carry = _csa(out[k], q, carry)
    return out


def _sc_body(x_hbm, posb_hbm, valb_hbm, out_hbm, xv, idx0, idx1, gval_v,
             posb_v, planes_v, out_v, sem):
    b = lax.axis_index("s") * 2 + lax.axis_index("c")

    pltpu.sync_copy(x_hbm.at[b], xv)

    for g in range(_P // 16):
        v = xv[pl.ds(g * 16, 16)] * jnp.float32(_L - 1)
        t = v.astype(jnp.int32)
        frac = v - t.astype(jnp.float32)
        half = jnp.float32(0.5)
        inc = jnp.where(
            (frac > half) | ((frac == half) & ((t & 1) == 1)), 1, 0)
        idx = jnp.minimum(jnp.maximum(t + inc, 0), _L - 1)
        if g < _PH0 // 16:
            idx0[pl.ds(g * 16, 16)] = idx
        else:
            idx1[pl.ds(g * 16 - _PH0, 16)] = idx

    zero = jnp.zeros((16,), jnp.uint32)

    for ph, (poff, prows, idxr) in enumerate(
            ((0, _PH0, idx0), (_PH0, _PH1, idx1))):
        gather = pltpu.async_copy(
            valb_hbm.at[idxr], gval_v.at[pl.ds(0, prows)], sem)
        pltpu.sync_copy(posb_hbm.at[pl.ds(poff, prows)],
                        posb_v.at[pl.ds(0, prows)])
        gather.wait()
        chunk = prows // _NCH

        def wg_body(wg, _, ph=ph, chunk=chunk):
            woff = wg * 16

            if ph == 0:
                planes0 = tuple(zero for _ in range(_PLANES))
            else:
                planes0 = tuple(
                    lax.bitcast_convert_type(
                        planes_v[pl.ds((wg * _PLANES + k) * 16, 16)],
                        jnp.uint32)
                    for k in range(_PLANES))

            def chunk_body(ci, planes):
                planes = list(planes)
                base = ci * chunk
                words = []
                for i in range(chunk):
                    vw = gval_v[base + i, pl.ds(woff, 16)]
                    pw = posb_v[base + i, pl.ds(woff, 16)]
                    words.append(
                        lax.bitcast_convert_type(vw ^ pw, jnp.uint32))
                return tuple(_merge_planes(planes, _reduce_chunk(words)))

            planes = lax.fori_loop(0, _NCH, chunk_body, planes0)

            if ph == 0:
                for k in range(_PLANES):
                    planes_v[pl.ds((wg * _PLANES + k) * 16, 16)] = (
                        lax.bitcast_convert_type(planes[k], jnp.int32))
            else:
                for j in range(32):
                    cnt = jnp.zeros((16,), jnp.uint32)
                    for k in range(_PLANES):
                        cnt = cnt + (((planes[k] >> j) & jnp.uint32(1)) << k)
                    out = jnp.where(cnt < jnp.uint32(_HALF_P),
                                    jnp.float32(1.0), jnp.float32(-1.0))
                    out_v[pl.ds(j * _W + wg * 16, 16)] = out
            return 0

        lax.fori_loop(0, _NWG, wg_body, 0)

    pltpu.sync_copy(out_v, out_hbm.at[b])


def kernel(x, position_weight, value_weight):
    xf = x.reshape(_B, _P)
    posbits = _pack_bits(position_weight)
    valbits = _pack_bits(value_weight)

    mesh = plsc.VectorSubcoreMesh(core_axis_name="c", subcore_axis_name="s")
    f = functools.partial(
        pl.kernel,
        out_type=jax.ShapeDtypeStruct((_B, _D), jnp.float32),
        mesh=mesh,
        compiler_params=pltpu.CompilerParams(needs_layout_passes=False),
        scratch_types=[
            pltpu.VMEM((_P,), jnp.float32),
            pltpu.VMEM((_PH0,), jnp.int32),
            pltpu.VMEM((_PH1,), jnp.int32),
            pltpu.VMEM((_PH0, _W), jnp.int32),
            pltpu.VMEM((_PH0, _W), jnp.int32),
            pltpu.VMEM((_NWG * _PLANES * 16,), jnp.int32),
            pltpu.VMEM((_D,), jnp.float32),
            pltpu.SemaphoreType.DMA,
        ],
    )(_sc_body)
    return f(xf, posbits, valbits)


# speedup vs baseline: 4.3916x; 1.1542x over previous
import functools

import jax
import jax.numpy as jnp
from jax import lax
from jax.experimental import pallas as pl
from jax.experimental.pallas import tpu as pltpu
from jax.experimental.pallas import tpu_sc as plsc

_D = 4096
_P = 784
_L = 256
_B = 32
_W = _D // 32
_PH0, _PH1 = 400, 384
_NWG = _W // 16
_NCH = 16
_HALF_P = _P // 2
_PLANES = 10


def _pack_bits(w):
    u = lax.bitcast_convert_type(w, jnp.uint32) >> 31
    acc = u[:, : _W]
    for j in range(1, 32):
        acc = acc | (u[:, j * _W:(j + 1) * _W] << j)
    return lax.bitcast_convert_type(acc, jnp.int32)


def _csa(a, b, c):
    t = a ^ b
    return t ^ c, (a & b) | (c & t)


def _reduce_chunk(words):
    pools = {}

    def push(k, x):
        pool = pools.setdefault(k, [])
        pool.append(x)
        if len(pool) >= 3:
            s, cy = _csa(pool.pop(), pool.pop(), pool.pop())
            pool.append(s)
            push(k + 1, cy)

    for x in words:
        push(0, x)
    for k in sorted(pools):
        pool = pools[k]
        while len(pool) > 1:
            a, b = pool.pop(), pool.pop()
            pool.append(a ^ b)
            push(k + 1, a & b)
    return {k: p[0] for k, p in pools.items() if p}


def _merge_planes(planes, addend):
    carry = None
    out = list(planes)
    for k in range(_PLANES):
        q = addend.get(k)
        if carry is None and q is None:
            continue
        if carry is None:
            out[k], carry = out[k] ^ q, out[k] & q
        elif q is None:
            out[k], carry = out[k] ^ carry, out[k] & carry
        else:
            out[k], carry = _csa(out[k], q, carry)
    return out


def _sc_body(x_hbm, posb_hbm, valb_hbm, out_hbm, xv, idx0, idx1, gval_v,
             posb_v, planes_v, out_v, sem):
    b = lax.axis_index("s") * 2 + lax.axis_index("c")

    pltpu.sync_copy(x_hbm.at[b], xv)

    for g in range(_P // 16):
        v = xv[pl.ds(g * 16, 16)] * jnp.float32(_L - 1)
        t = v.astype(jnp.int32)
        frac = v - t.astype(jnp.float32)
        half = jnp.float32(0.5)
        inc = jnp.where(
            (frac > half) | ((frac == half) & ((t & 1) == 1)), 1, 0)
        idx = jnp.minimum(jnp.maximum(t + inc, 0), _L - 1)
        if g < _PH0 // 16:
            idx0[pl.ds(g * 16, 16)] = idx
        else:
            idx1[pl.ds(g * 16 - _PH0, 16)] = idx

    zero = jnp.zeros((16,), jnp.uint32)

    for ph, (poff, prows, idxr) in enumerate(
            ((0, _PH0, idx0), (_PH0, _PH1, idx1))):
        gather = pltpu.async_copy(
            valb_hbm.at[idxr], gval_v.at[pl.ds(0, prows)], sem)
        pltpu.sync_copy(posb_hbm.at[pl.ds(poff, prows)],
                        posb_v.at[pl.ds(0, prows)])
        gather.wait()
        chunk = prows // _NCH

        for wg in range(_NWG):
            woff = wg * 16

            if ph == 0:
                planes0 = tuple(zero for _ in range(_PLANES))
            else:
                planes0 = tuple(
                    lax.bitcast_convert_type(
                        planes_v[(wg * _PLANES + k) * 16:
                                 (wg * _PLANES + k) * 16 + 16],
                        jnp.uint32)
                    for k in range(_PLANES))

            def chunk_body(ci, planes, chunk=chunk, woff=woff):
                base = ci * chunk
                pools = {}

                def push(k, x):
                    pool = pools.setdefault(k, [])
                    pool.append(x)
                    if len(pool) >= 3:
                        s, cy = _csa(pool.pop(), pool.pop(), pool.pop())
                        pool.append(s)
                        push(k + 1, cy)

                for i in range(chunk):
                    vw = gval_v[base + i, woff:woff + 16]
                    pw = posb_v[base + i, woff:woff + 16]
                    push(0, lax.bitcast_convert_type(vw ^ pw, jnp.uint32))
                for k in sorted(pools):
                    pool = pools[k]
                    while len(pool) > 1:
                        a, c = pool.pop(), pool.pop()
                        pool.append(a ^ c)
                        push(k + 1, a & c)
                addend = {k: p[0] for k, p in pools.items() if p}
                return tuple(_merge_planes(planes, addend))

            planes = lax.fori_loop(0, _NCH, chunk_body, planes0)

            if ph == 0:
                for k in range(_PLANES):
                    planes_v[(wg * _PLANES + k) * 16:
                             (wg * _PLANES + k) * 16 + 16] = (
                        lax.bitcast_convert_type(planes[k], jnp.int32))
            else:
                p = planes
                c7 = p[6] | p[5] | p[4] | p[3]
                ge = p[9] | (p[8] & p[7] & c7)
                for j in range(32):
                    bit = (ge >> j) & jnp.uint32(1)
                    out = jnp.where(bit == 0, jnp.float32(1.0),
                                    jnp.float32(-1.0))
                    out_v[j * _W + woff:j * _W + woff + 16] = out

    pltpu.sync_copy(out_v, out_hbm.at[b])


def kernel(x, position_weight, value_weight):
    xf = x.reshape(_B, _P)
    posbits = _pack_bits(position_weight)
    valbits = _pack_bits(value_weight)

    mesh = plsc.VectorSubcoreMesh(core_axis_name="c", subcore_axis_name="s")
    f = functools.partial(
        pl.kernel,
        out_type=jax.ShapeDtypeStruct((_B, _D), jnp.float32),
        mesh=mesh,
        compiler_params=pltpu.CompilerParams(needs_layout_passes=False),
        scratch_types=[
            pltpu.VMEM((_P,), jnp.float32),
            pltpu.VMEM((_PH0,), jnp.int32),
            pltpu.VMEM((_PH1,), jnp.int32),
            pltpu.VMEM((_PH0, _W), jnp.int32),
            pltpu.VMEM((_PH0, _W), jnp.int32),
            pltpu.VMEM((_NWG * _PLANES * 16,), jnp.int32),
            pltpu.VMEM((_D,), jnp.float32),
            pltpu.SemaphoreType.DMA,
        ],
    )(_sc_body)
    return f(xf, posbits, valbits)


# speedup vs baseline: 4.6289x; 1.0540x over previous
import functools

import jax
import jax.numpy as jnp
from jax import lax
from jax.experimental import pallas as pl
from jax.experimental.pallas import tpu as pltpu
from jax.experimental.pallas import tpu_sc as plsc

_D = 4096
_P = 784
_L = 256
_B = 32
_W = _D // 32
_PH0, _PH1 = 400, 384
_NWG = _W // 16
_NCH = 8
_HALF_P = _P // 2
_PLANES = 10


def _pack_body(w_ref, out_ref):
    u = lax.bitcast_convert_type(w_ref[...], jnp.uint32) >> 31
    acc = u[:, : _W]
    for j in range(1, 32):
        acc = acc | (u[:, j * _W:(j + 1) * _W] << j)
    out_ref[...] = lax.bitcast_convert_type(acc, jnp.int32)


def _pack_bits(w):
    n = w.shape[0]
    blk = 56 if n % 56 == 0 else 32
    return pl.pallas_call(
        _pack_body,
        grid=(n // blk,),
        in_specs=[pl.BlockSpec((blk, _D), lambda i: (i, 0))],
        out_specs=pl.BlockSpec((blk, _W), lambda i: (i, 0)),
        out_shape=jax.ShapeDtypeStruct((n, _W), jnp.int32),
    )(w)


def _csa(a, b, c):
    t = a ^ b
    return t ^ c, (a & b) | (c & t)


def _reduce_chunk(words):
    pools = {}

    def push(k, x):
        pool = pools.setdefault(k, [])
        pool.append(x)
        if len(pool) >= 3:
            s, cy = _csa(pool.pop(), pool.pop(), pool.pop())
            pool.append(s)
            push(k + 1, cy)

    for x in words:
        push(0, x)
    for k in sorted(pools):
        pool = pools[k]
        while len(pool) > 1:
            a, b = pool.pop(), pool.pop()
            pool.append(a ^ b)
            push(k + 1, a & b)
    return {k: p[0] for k, p in pools.items() if p}


def _merge_planes(planes, addend):
    carry = None
    out = list(planes)
    for k in range(_PLANES):
        q = addend.get(k)
        if carry is None and q is None:
            continue
        if carry is None:
            out[k], carry = out[k] ^ q, out[k] & q
        elif q is None:
            out[k], carry = out[k] ^ carry, out[k] & carry
        else:
            out[k], carry = _csa(out[k], q, carry)
    return out


def _sc_body(x_hbm, posb_hbm, valb_hbm, out_hbm, xv, idx0, idx1, gval_v,
             posb_v, planes_v, out_v, sem):
    b = lax.axis_index("s") * 2 + lax.axis_index("c")

    pltpu.sync_copy(x_hbm.at[b], xv)

    for g in range(_P // 16):
        v = xv[pl.ds(g * 16, 16)] * jnp.float32(_L - 1)
        t = v.astype(jnp.int32)
        frac = v - t.astype(jnp.float32)
        half = jnp.float32(0.5)
        inc = jnp.where(
            (frac > half) | ((frac == half) & ((t & 1) == 1)), 1, 0)
        idx = jnp.minimum(jnp.maximum(t + inc, 0), _L - 1)
        if g < _PH0 // 16:
            idx0[pl.ds(g * 16, 16)] = idx
        else:
            idx1[pl.ds(g * 16 - _PH0, 16)] = idx

    zero = jnp.zeros((16,), jnp.uint32)

    for ph, (poff, prows, idxr) in enumerate(
            ((0, _PH0, idx0), (_PH0, _PH1, idx1))):
        gather = pltpu.async_copy(
            valb_hbm.at[idxr], gval_v.at[pl.ds(0, prows)], sem)
        pltpu.sync_copy(posb_hbm.at[pl.ds(poff, prows)],
                        posb_v.at[pl.ds(0, prows)])
        gather.wait()
        chunk = prows // _NCH

        for wg in range(_NWG):
            woff = wg * 16

            if ph == 0:
                planes0 = tuple(zero for _ in range(_PLANES))
            else:
                planes0 = tuple(
                    lax.bitcast_convert_type(
                        planes_v[(wg * _PLANES + k) * 16:
                                 (wg * _PLANES + k) * 16 + 16],
                        jnp.uint32)
                    for k in range(_PLANES))

            def chunk_body(ci, planes, chunk=chunk, woff=woff):
                base = ci * chunk
                pools = {}

                def push(k, x):
                    pool = pools.setdefault(k, [])
                    pool.append(x)
                    if len(pool) >= 3:
                        s, cy = _csa(pool.pop(), pool.pop(), pool.pop())
                        pool.append(s)
                        push(k + 1, cy)

                for i in range(chunk):
                    vw = gval_v[base + i, woff:woff + 16]
                    pw = posb_v[base + i, woff:woff + 16]
                    push(0, lax.bitcast_convert_type(vw ^ pw, jnp.uint32))
                for k in sorted(pools):
                    pool = pools[k]
                    while len(pool) > 1:
                        a, c = pool.pop(), pool.pop()
                        pool.append(a ^ c)
                        push(k + 1, a & c)
                addend = {k: p[0] for k, p in pools.items() if p}
                return tuple(_merge_planes(planes, addend))

            planes = lax.fori_loop(0, _NCH, chunk_body, planes0)

            if ph == 0:
                for k in range(_PLANES):
                    planes_v[(wg * _PLANES + k) * 16:
                             (wg * _PLANES + k) * 16 + 16] = (
                        lax.bitcast_convert_type(planes[k], jnp.int32))
            else:
                p = planes
                c7 = p[6] | p[5] | p[4] | p[3]
                ge = p[9] | (p[8] & p[7] & c7)
                for j in range(32):
                    bit = (ge >> j) & jnp.uint32(1)
                    out = jnp.where(bit == 0, jnp.float32(1.0),
                                    jnp.float32(-1.0))
                    out_v[j * _W + woff:j * _W + woff + 16] = out

    pltpu.sync_copy(out_v, out_hbm.at[b])


def kernel(x, position_weight, value_weight):
    xf = x.reshape(_B, _P)
    posbits = _pack_bits(position_weight)
    valbits = _pack_bits(value_weight)

    mesh = plsc.VectorSubcoreMesh(core_axis_name="c", subcore_axis_name="s")
    f = functools.partial(
        pl.kernel,
        out_type=jax.ShapeDtypeStruct((_B, _D), jnp.float32),
        mesh=mesh,
        compiler_params=pltpu.CompilerParams(needs_layout_passes=False),
        scratch_types=[
            pltpu.VMEM((_P,), jnp.float32),
            pltpu.VMEM((_PH0,), jnp.int32),
            pltpu.VMEM((_PH1,), jnp.int32),
            pltpu.VMEM((_PH0, _W), jnp.int32),
            pltpu.VMEM((_PH0, _W), jnp.int32),
            pltpu.VMEM((_NWG * _PLANES * 16,), jnp.int32),
            pltpu.VMEM((_D,), jnp.float32),
            pltpu.SemaphoreType.DMA,
        ],
    )(_sc_body)
    return f(xf, posbits, valbits)


# speedup vs baseline: 5.0071x; 1.0817x over previous
import functools

import jax
import jax.numpy as jnp
from jax import lax
from jax.experimental import pallas as pl
from jax.experimental.pallas import tpu as pltpu
from jax.experimental.pallas import tpu_sc as plsc

_D = 4096
_P = 784
_L = 256
_B = 32
_W = _D // 32
_PHASES = (208, 192, 192, 192)
_POFF = (0, 208, 400, 592)
_NWG = _W // 16
_NCH = 8
_HALF_P = _P // 2
_PLANES = 10


def _pack_body(w_ref, out_ref):
    u = lax.bitcast_convert_type(w_ref[...], jnp.uint32) >> 31
    acc = u[:, : _W]
    for j in range(1, 32):
        acc = acc | (u[:, j * _W:(j + 1) * _W] << j)
    out_ref[...] = lax.bitcast_convert_type(acc, jnp.int32)


def _pack_bits(w):
    n = w.shape[0]
    blk = 56 if n % 56 == 0 else 32
    return pl.pallas_call(
        _pack_body,
        grid=(n // blk,),
        in_specs=[pl.BlockSpec((blk, _D), lambda i: (i, 0))],
        out_specs=pl.BlockSpec((blk, _W), lambda i: (i, 0)),
        out_shape=jax.ShapeDtypeStruct((n, _W), jnp.int32),
    )(w)


def _csa(a, b, c):
    t = a ^ b
    return t ^ c, (a & b) | (c & t)


def _reduce_chunk(words):
    pools = {}

    def push(k, x):
        pool = pools.setdefault(k, [])
        pool.append(x)
        if len(pool) >= 3:
            s, cy = _csa(pool.pop(), pool.pop(), pool.pop())
            pool.append(s)
            push(k + 1, cy)

    for x in words:
        push(0, x)
    for k in sorted(pools):
        pool = pools[k]
        while len(pool) > 1:
            a, b = pool.pop(), pool.pop()
            pool.append(a ^ b)
            push(k + 1, a & b)
    return {k: p[0] for k, p in pools.items() if p}


def _merge_planes(planes, addend):
    carry = None
    out = list(planes)
    for k in range(_PLANES):
        q = addend.get(k)
        if carry is None and q is None:
            continue
        if carry is None:
            out[k], carry = out[k] ^ q, out[k] & q
        elif q is None:
            out[k], carry = out[k] ^ carry, out[k] & carry
        else:
            out[k], carry = _csa(out[k], q, carry)
    return out


def _sc_body(x_hbm, posb_hbm, valb_hbm, out_hbm, xv, idxv, gval0, gval1,
             posb0, posb1, planes_v, out_v, sg0, sg1, sp0, sp1):
    b = lax.axis_index("s") * 2 + lax.axis_index("c")

    pltpu.sync_copy(x_hbm.at[b], xv)

    for g in range(_P // 16):
        v = xv[pl.ds(g * 16, 16)] * jnp.float32(_L - 1)
        t = v.astype(jnp.int32)
        frac = v - t.astype(jnp.float32)
        half = jnp.float32(0.5)
        inc = jnp.where(
            (frac > half) | ((frac == half) & ((t & 1) == 1)), 1, 0)
        idx = jnp.minimum(jnp.maximum(t + inc, 0), _L - 1)
        idxv[pl.ds(g * 16, 16)] = idx

    zero = jnp.zeros((16,), jnp.uint32)
    gval = (gval0, gval1)
    posb = (posb0, posb1)
    sems = ((sg0, sp0), (sg1, sp1))

    def start_phase(ph):
        buf = ph % 2
        rows = _PHASES[ph]
        cp0 = pltpu.async_copy(
            valb_hbm.at[idxv.at[pl.ds(_POFF[ph], rows)]],
            gval[buf].at[pl.ds(0, rows)], sems[buf][0])
        cp1 = pltpu.async_copy(
            posb_hbm.at[pl.ds(_POFF[ph], rows)],
            posb[buf].at[pl.ds(0, rows)], sems[buf][1])
        return cp0, cp1

    pend = start_phase(0)

    for ph, prows in enumerate(_PHASES):
        buf = ph % 2
        for cp in pend:
            cp.wait()
        if ph + 1 < len(_PHASES):
            pend = start_phase(ph + 1)
        gval_v = gval[buf]
        posb_v = posb[buf]
        chunk = prows // _NCH

        for wg in range(_NWG):
            woff = wg * 16

            if ph == 0:
                planes0 = tuple(zero for _ in range(_PLANES))
            else:
                planes0 = tuple(
                    lax.bitcast_convert_type(
                        planes_v[(wg * _PLANES + k) * 16:
                                 (wg * _PLANES + k) * 16 + 16],
                        jnp.uint32)
                    for k in range(_PLANES))

            def chunk_body(ci, planes, chunk=chunk, woff=woff,
                           gval_v=gval_v, posb_v=posb_v):
                base = ci * chunk
                pools = {}

                def push(k, x):
                    pool = pools.setdefault(k, [])
                    pool.append(x)
                    if len(pool) >= 3:
                        s, cy = _csa(pool.pop(), pool.pop(), pool.pop())
                        pool.append(s)
                        push(k + 1, cy)

                for i in range(chunk):
                    vw = gval_v[base + i, woff:woff + 16]
                    pw = posb_v[base + i, woff:woff + 16]
                    push(0, lax.bitcast_convert_type(vw ^ pw, jnp.uint32))
                for k in sorted(pools):
                    pool = pools[k]
                    while len(pool) > 1:
                        a, c = pool.pop(), pool.pop()
                        pool.append(a ^ c)
                        push(k + 1, a & c)
                addend = {k: p[0] for k, p in pools.items() if p}
                return tuple(_merge_planes(planes, addend))

            planes = lax.fori_loop(0, _NCH, chunk_body, planes0)

            if ph + 1 < len(_PHASES):
                for k in range(_PLANES):
                    planes_v[(wg * _PLANES + k) * 16:
                             (wg * _PLANES + k) * 16 + 16] = (
                        lax.bitcast_convert_type(planes[k], jnp.int32))
            else:
                p = planes
                c7 = p[6] | p[5] | p[4] | p[3]
                ge = p[9] | (p[8] & p[7] & c7)
                for j in range(32):
                    bit = (ge >> j) & jnp.uint32(1)
                    out = jnp.where(bit == 0, jnp.float32(1.0),
                                    jnp.float32(-1.0))
                    out_v[j * _W + woff:j * _W + woff + 16] = out

    pltpu.sync_copy(out_v, out_hbm.at[b])


def kernel(x, position_weight, value_weight):
    xf = x.reshape(_B, _P)
    posbits = _pack_bits(position_weight)
    valbits = _pack_bits(value_weight)

    mesh = plsc.VectorSubcoreMesh(core_axis_name="c", subcore_axis_name="s")
    f = functools.partial(
        pl.kernel,
        out_type=jax.ShapeDtypeStruct((_B, _D), jnp.float32),
        mesh=mesh,
        compiler_params=pltpu.CompilerParams(needs_layout_passes=False),
        scratch_types=[
            pltpu.VMEM((_P,), jnp.float32),
            pltpu.VMEM((_P,), jnp.int32),
            pltpu.VMEM((_PHASES[0], _W), jnp.int32),
            pltpu.VMEM((_PHASES[0], _W), jnp.int32),
            pltpu.VMEM((_PHASES[0], _W), jnp.int32),
            pltpu.VMEM((_PHASES[0], _W), jnp.int32),
            pltpu.VMEM((_NWG * _PLANES * 16,), jnp.int32),
            pltpu.VMEM((_D,), jnp.float32),
            pltpu.SemaphoreType.DMA,
            pltpu.SemaphoreType.DMA,
            pltpu.SemaphoreType.DMA,
            pltpu.SemaphoreType.DMA,
        ],
    )(_sc_body)
    return f(xf, posbits, valbits)
